# baseline (speedup 1.0000x reference)
import jax
import jax.numpy as jnp
from jax import lax
from jax.experimental import pallas as pl
from jax.experimental.pallas import tpu as pltpu
from jax.experimental.pallas import tpu_sc as plsc

B, H, D = 4096, 200, 64
NV = 1000
ROWS = B * H
GR = 2 * ROWS
NC, NS = 2, 16
NW = NC * NS
GPW = GR // NW
C = 128
IROWS = GR // C
IRPW = IROWS // NW
NBUF = 4
T = IRPW // NBUF


def _body(idx_hbm, table_hbm, out_hbm,
          idx_v, b0, b1, b2, b3, sem_g, sem_w):
    bufs = [b0, b1, b2, b3]
    wid = lax.axis_index("s") * NC + lax.axis_index("c")
    grow0 = wid * GPW

    pltpu.sync_copy(idx_hbm.at[pl.ds(wid * IRPW, IRPW)], idx_v)

    def wait_write(j):
        pltpu.make_async_copy(
            bufs[j], out_hbm.at[pl.ds(0, C)], sem_w.at[j]).wait()

    def block(t, carry):
        @pl.when(t > 0)
        def _():
            for j in range(NBUF):
                wait_write(j)
        cps = []
        for j in range(NBUF):
            g = t * NBUF + j
            cps.append(pltpu.async_copy(
                table_hbm.at[idx_v.at[g]], bufs[j], sem_g.at[j]))
        for j in range(NBUF):
            g = t * NBUF + j
            cps[j].wait()
            pltpu.async_copy(
                bufs[j], out_hbm.at[pl.ds(grow0 + g * C, C)], sem_w.at[j])
        return carry

    lax.fori_loop(0, T, block, 0)
    for j in range(NBUF):
        wait_write(j)


@jax.jit
def _run(idx, table):
    fn = pl.kernel(
        _body,
        out_type=jax.ShapeDtypeStruct((GR, D), jnp.float32),
        mesh=plsc.VectorSubcoreMesh(core_axis_name="c", subcore_axis_name="s"),
        compiler_params=pltpu.CompilerParams(
            use_tc_tiling_on_sc=False, needs_layout_passes=False),
        scratch_types=(
            [pltpu.VMEM((IRPW, C), jnp.int32)]
            + [pltpu.VMEM((C, D), jnp.float32)] * NBUF
            + [pltpu.SemaphoreType.DMA((NBUF,))] * 2
        ),
    )
    return fn(idx, table)


def kernel(observed_labels, verbs_table, nouns_table):
    table = jnp.concatenate([verbs_table, nouns_table], axis=0)
    idx = (observed_labels.reshape(ROWS, 2)
           + jnp.array([0, NV], jnp.int32)).reshape(IROWS, C)
    out = _run(idx, table)
    return out.reshape(B, H, 2 * D)


# speedup vs baseline: 3.0255x; 3.0255x over previous
import jax
import jax.numpy as jnp
from jax import lax
from jax.experimental import pallas as pl
from jax.experimental.pallas import tpu as pltpu
from jax.experimental.pallas import tpu_sc as plsc

B, H, D = 4096, 200, 64
ROWS = B * H
NC, NS = 2, 16
NW = NC * NS
RPW = ROWS // NW
C = 128
IROWS = ROWS // C
IRPW = IROWS // NW
NBUF = 4
T = IRPW // NBUF


def _body(idx_hbm, verbs_hbm, nouns_hbm, out_hbm,
          vidx_v, nidx_v,
          vb0, vb1, vb2, vb3, nb0, nb1, nb2, nb3,
          sem_gv, sem_gn, sem_wv, sem_wn):
    vbufs = [vb0, vb1, vb2, vb3]
    nbufs = [nb0, nb1, nb2, nb3]
    wid = lax.axis_index("s") * NC + lax.axis_index("c")
    row0 = wid * RPW

    pltpu.sync_copy(idx_hbm.at[0, pl.ds(wid * IRPW, IRPW)], vidx_v)
    pltpu.sync_copy(idx_hbm.at[1, pl.ds(wid * IRPW, IRPW)], nidx_v)

    def wait_write_v(j):
        pltpu.make_async_copy(
            vbufs[j], out_hbm.at[pl.ds(0, C), pl.ds(0, D)], sem_wv.at[j]).wait()

    def wait_write_n(j):
        pltpu.make_async_copy(
            nbufs[j], out_hbm.at[pl.ds(0, C), pl.ds(D, D)], sem_wn.at[j]).wait()

    def block(t, carry):
        @pl.when(t > 0)
        def _():
            for j in range(NBUF):
                wait_write_v(j)
                wait_write_n(j)
        cps = []
        for j in range(NBUF):
            g = t * NBUF + j
            cps.append((
                pltpu.async_copy(verbs_hbm.at[vidx_v.at[g]], vbufs[j], sem_gv.at[j]),
                pltpu.async_copy(nouns_hbm.at[nidx_v.at[g]], nbufs[j], sem_gn.at[j]),
            ))
        for j in range(NBUF):
            g = t * NBUF + j
            r0 = row0 + g * C
            cps[j][0].wait()
            cps[j][1].wait()
            pltpu.async_copy(vbufs[j], out_hbm.at[pl.ds(r0, C), pl.ds(0, D)], sem_wv.at[j])
            pltpu.async_copy(nbufs[j], out_hbm.at[pl.ds(r0, C), pl.ds(D, D)], sem_wn.at[j])
        return carry

    lax.fori_loop(0, T, block, 0)
    for j in range(NBUF):
        wait_write_v(j)
        wait_write_n(j)


@jax.jit
def _run(idx, verbs_table, nouns_table):
    fn = pl.kernel(
        _body,
        out_type=jax.ShapeDtypeStruct((ROWS, 2 * D), jnp.float32),
        mesh=plsc.VectorSubcoreMesh(core_axis_name="c", subcore_axis_name="s"),
        compiler_params=pltpu.CompilerParams(
            use_tc_tiling_on_sc=False, needs_layout_passes=False),
        scratch_types=(
            [pltpu.VMEM((IRPW, C), jnp.int32)] * 2
            + [pltpu.VMEM((C, D), jnp.float32)] * (2 * NBUF)
            + [pltpu.SemaphoreType.DMA((NBUF,))] * 4
        ),
    )
    return fn(idx, verbs_table, nouns_table)


def kernel(observed_labels, verbs_table, nouns_table):
    idx = observed_labels.reshape(ROWS, 2).T.reshape(2, IROWS, C)
    out = _run(idx, verbs_table, nouns_table)
    return out.reshape(B, H, 2 * D)
